# Initial kernel scaffold; baseline (speedup 1.0000x reference)
import jax
import jax.numpy as jnp
from jax.experimental import pallas as pl


def kernel(x, edge_index, batch, W1_0, b1_0, W2_0, b2_0, gamma_0, beta_0, W1_1, b1_1, W2_1, b2_1, gamma_1, beta_1):
    raise NotImplementedError("write your pallas kernel here")


# speedup vs baseline: 4.0964x; 4.0964x over previous
import functools

import jax
import jax.numpy as jnp
from jax import lax
from jax.experimental import pallas as pl
from jax.experimental.pallas import tpu as pltpu
from jax.experimental.pallas import tpu_sc as plsc

N = 10000
E = 320000
D = 128
G = 64
BN_EPS = 1e-5

NC = 2
NS = 16
NW = NC * NS
EPW = E // NW
CH = 80
NCHUNK = EPW // CH
N_PAD = 10240
RPT = N_PAD // NS
RCH = RPT // CH

RB = 400
NBLK = N // RB


def _sc_segsum_body(h_hbm, src_hbm, dst_hbm, zeros_hbm, out_hbm,
                    src_v, dst_v, rows_v, acc_sh, sem):
    c = lax.axis_index("c")
    s = lax.axis_index("s")
    wid = s * NC + c

    pltpu.sync_copy(zeros_hbm, rows_v)

    def zbody(j, carry):
        pltpu.sync_copy(rows_v, acc_sh.at[pl.ds(s * RPT + j * CH, CH)])
        return carry

    lax.fori_loop(0, RCH, zbody, 0)
    plsc.subcore_barrier()

    def body(i, carry):
        base = pl.multiple_of(wid * EPW + i * CH, 8)
        pltpu.sync_copy(src_hbm.at[pl.ds(base, CH)], src_v)
        pltpu.sync_copy(dst_hbm.at[pl.ds(base, CH)], dst_v)
        pltpu.async_copy(h_hbm.at[src_v], rows_v, sem).wait()
        pltpu.sync_copy(rows_v, acc_sh.at[dst_v], add=True)
        return carry

    lax.fori_loop(0, NCHUNK, body, 0)
    plsc.subcore_barrier()

    def obody(j, carry):
        off = s * RPT + j * CH
        pltpu.sync_copy(acc_sh.at[pl.ds(off, CH)], rows_v)
        pltpu.sync_copy(rows_v, out_hbm.at[c, pl.ds(off, CH)])
        return carry

    lax.fori_loop(0, RCH, obody, 0)


@functools.lru_cache(maxsize=1)
def _sc_segsum_kernel():
    return pl.kernel(
        _sc_segsum_body,
        out_type=jax.ShapeDtypeStruct((NC, N_PAD, D), jnp.float32),
        mesh=plsc.VectorSubcoreMesh(
            core_axis_name="c", subcore_axis_name="s",
            num_cores=NC, num_subcores=NS),
        scratch_types=[
            pltpu.VMEM((CH,), jnp.int32),
            pltpu.VMEM((CH,), jnp.int32),
            pltpu.VMEM((CH, D), jnp.float32),
            pltpu.VMEM_SHARED((N_PAD, D), jnp.float32),
            pltpu.SemaphoreType.DMA,
        ],
    )


def _sc_segsum(h, src, dst, zeros):
    return _sc_segsum_kernel()(h, src, dst, zeros)


def _mlp_body(h_ref, p0_ref, p1_ref, w1_ref, b1_ref, w2_ref, b2_ref,
              z2_ref, st_ref):
    i = pl.program_id(0)
    z = h_ref[...] + p0_ref[...] + p1_ref[...]
    a = jnp.maximum(
        jnp.dot(z, w1_ref[...], preferred_element_type=jnp.float32)
        + b1_ref[...], 0.0)
    z2 = (jnp.dot(a, w2_ref[...], preferred_element_type=jnp.float32)
          + b2_ref[...])
    z2_ref[...] = z2
    s0 = jnp.sum(z2, axis=0, keepdims=True)
    s1 = jnp.sum(z2 * z2, axis=0, keepdims=True)
    contrib = jnp.concatenate(
        [s0, s1, jnp.zeros((6, D), jnp.float32)], axis=0)

    @pl.when(i == 0)
    def _():
        st_ref[...] = contrib

    @pl.when(i > 0)
    def _():
        st_ref[...] = st_ref[...] + contrib


def _mlp(h, p0, p1, W1, b1, W2, b2):
    return pl.pallas_call(
        _mlp_body,
        grid=(NBLK,),
        in_specs=[
            pl.BlockSpec((RB, D), lambda i: (i, 0)),
            pl.BlockSpec((RB, D), lambda i: (i, 0)),
            pl.BlockSpec((RB, D), lambda i: (i, 0)),
            pl.BlockSpec((D, 2 * D), lambda i: (0, 0)),
            pl.BlockSpec((1, 2 * D), lambda i: (0, 0)),
            pl.BlockSpec((2 * D, D), lambda i: (0, 0)),
            pl.BlockSpec((1, D), lambda i: (0, 0)),
        ],
        out_specs=[
            pl.BlockSpec((RB, D), lambda i: (i, 0)),
            pl.BlockSpec((8, D), lambda i: (0, 0)),
        ],
        out_shape=[
            jax.ShapeDtypeStruct((N, D), jnp.float32),
            jax.ShapeDtypeStruct((8, D), jnp.float32),
        ],
    )(h, p0, p1, W1, b1.reshape(1, 2 * D), W2, b2.reshape(1, D))


def _bnrelu_body(z_ref, sc_ref, sh_ref, o_ref):
    o_ref[...] = jnp.maximum(z_ref[...] * sc_ref[...] + sh_ref[...], 0.0)


def _bnrelu(z, scale, shift):
    return pl.pallas_call(
        _bnrelu_body,
        grid=(NBLK,),
        in_specs=[
            pl.BlockSpec((RB, D), lambda i: (i, 0)),
            pl.BlockSpec((1, D), lambda i: (0, 0)),
            pl.BlockSpec((1, D), lambda i: (0, 0)),
        ],
        out_specs=pl.BlockSpec((RB, D), lambda i: (i, 0)),
        out_shape=jax.ShapeDtypeStruct((N, D), jnp.float32),
    )(z, scale, shift)


def _segmax_body(starts_ref, blo_ref, bhi_ref, z_ref, sc_ref, sh_ref, out_ref):
    b = pl.program_id(0)

    @pl.when(b == 0)
    def _():
        out_ref[...] = jnp.full((G, D), -jnp.inf, jnp.float32)

    zn = z_ref[...] * sc_ref[...] + sh_ref[...]
    gi = b * RB + lax.broadcasted_iota(jnp.int32, (RB, D), 0)
    gidx = lax.broadcasted_iota(jnp.int32, (G, D), 0)

    def gbody(g, carry):
        lo = starts_ref[g]
        hi = starts_ref[g + 1]
        m = (gi >= lo) & (gi < hi)
        v = jnp.max(jnp.where(m, zn, -jnp.inf), axis=0, keepdims=True)
        upd = jnp.where(gidx == g, v, -jnp.inf)
        out_ref[...] = jnp.maximum(out_ref[...], upd)
        return carry

    lax.fori_loop(blo_ref[b], bhi_ref[b] + 1, gbody, 0)


def _segmax(z, scale, shift, starts, blo, bhi):
    return pl.pallas_call(
        _segmax_body,
        grid=(NBLK,),
        in_specs=[
            pl.BlockSpec(memory_space=pltpu.SMEM),
            pl.BlockSpec(memory_space=pltpu.SMEM),
            pl.BlockSpec(memory_space=pltpu.SMEM),
            pl.BlockSpec((RB, D), lambda i: (i, 0)),
            pl.BlockSpec((1, D), lambda i: (0, 0)),
            pl.BlockSpec((1, D), lambda i: (0, 0)),
        ],
        out_specs=pl.BlockSpec((G, D), lambda i: (0, 0)),
        out_shape=jax.ShapeDtypeStruct((G, D), jnp.float32),
    )(starts, blo, bhi, z, scale, shift)


def kernel(x, edge_index, batch,
           W1_0, b1_0, W2_0, b2_0, gamma_0, beta_0,
           W1_1, b1_1, W2_1, b2_1, gamma_1, beta_1):
    src = edge_index[0]
    dst = edge_index[1]
    zeros = jnp.zeros((CH, D), jnp.float32)
    starts = jnp.searchsorted(
        batch, jnp.arange(G + 1, dtype=jnp.int32), side='left'
    ).astype(jnp.int32)
    bidx = jnp.arange(NBLK, dtype=jnp.int32)
    blo = batch[bidx * RB]
    bhi = batch[bidx * RB + RB - 1]

    W1s = jnp.stack([W1_0, W1_1])
    b1s = jnp.stack([b1_0, b1_1])
    W2s = jnp.stack([W2_0, W2_1])
    b2s = jnp.stack([b2_0, b2_1])
    gammas = jnp.stack([gamma_0, gamma_1])
    betas = jnp.stack([beta_0, beta_1])

    def layer(h, p):
        W1, b1, W2, b2, gamma, beta = p
        parts = _sc_segsum(h, src, dst, zeros)
        z2, st = _mlp(h, parts[0, :N], parts[1, :N], W1, b1, W2, b2)
        mean = st[0] / N
        var = st[1] / N - mean * mean
        scale = (gamma * lax.rsqrt(var + BN_EPS)).reshape(1, D)
        shift = (beta - mean * scale[0]).reshape(1, D)
        h_next = _bnrelu(z2, scale, shift)
        return h_next, (z2, scale, shift)

    _, (z2s, scales, shifts) = lax.scan(
        layer, x, (W1s, b1s, W2s, b2s, gammas, betas))

    return _segmax(z2s[1], scales[1], shifts[1], starts, blo, bhi)


# speedup vs baseline: 6.7365x; 1.6445x over previous
import functools

import jax
import jax.numpy as jnp
from jax import lax
from jax.experimental import pallas as pl
from jax.experimental.pallas import tpu as pltpu
from jax.experimental.pallas import tpu_sc as plsc

N = 10000
E = 320000
D = 128
G = 64
BN_EPS = 1e-5

NC = 2
NS = 16
NW = NC * NS
EPW = E // NW
CH = 80
NCHUNK = EPW // CH
N_PAD = 10240
RPT = N_PAD // NS
RCH = RPT // CH

RB = 400
NBLK = N // RB


def _sc_segsum_body(h_hbm, src_hbm, dstc_hbm, zeros_hbm, out_hbm,
                    src_v, dst_v, rows0_v, rows1_v, acc_sh,
                    isem, gsem0, gsem1, ssem0, ssem1):
    c = lax.axis_index("c")
    s = lax.axis_index("s")
    wid = s * NC + c

    rows = (rows0_v, rows1_v)
    gsem = (gsem0, gsem1)
    ssem = (ssem0, ssem1)

    src_view = src_hbm.at[pl.ds(pl.multiple_of(wid * EPW, 8), EPW)]
    dst_view = dstc_hbm.at[wid]
    pltpu.async_copy(src_view, src_v, isem)
    pltpu.async_copy(dst_view, dst_v, isem)

    pltpu.sync_copy(zeros_hbm, rows0_v)

    def zbody(j, carry):
        pltpu.sync_copy(rows0_v, acc_sh.at[pl.ds(s * RPT + j * CH, CH)])
        return carry

    lax.fori_loop(0, RCH, zbody, 0)
    pltpu.make_async_copy(src_view, src_v, isem).wait()
    pltpu.make_async_copy(dst_view, dst_v, isem).wait()
    plsc.subcore_barrier()

    def gather(i, p):
        pltpu.async_copy(h_hbm.at[src_v.at[pl.ds(i * CH, CH)]],
                         rows[p], gsem[p])

    def scat_start(i, p):
        pltpu.async_copy(rows[p], acc_sh.at[dst_v.at[i]], ssem[p],
                         add=True)

    def scat_wait(p):
        pltpu.make_async_copy(rows[p], acc_sh.at[pl.ds(0, CH)],
                              ssem[p]).wait()

    def gat_wait(p):
        pltpu.make_async_copy(h_hbm.at[pl.ds(0, CH)], rows[p],
                              gsem[p]).wait()

    gather(0, 0)
    gather(1, 1)

    def body(ii, carry):
        i = ii * 2
        gat_wait(0)
        scat_start(i, 0)
        gat_wait(1)
        scat_start(i + 1, 1)
        scat_wait(0)
        gather(i + 2, 0)
        scat_wait(1)

        @pl.when(i + 3 < NCHUNK)
        def _():
            gather(i + 3, 1)

        return carry

    lax.fori_loop(0, (NCHUNK - 1) // 2, body, 0)
    gat_wait(0)
    scat_start(NCHUNK - 1, 0)
    scat_wait(0)
    plsc.subcore_barrier()

    def obody(j, carry):
        off = s * RPT + j * CH
        pltpu.sync_copy(acc_sh.at[pl.ds(off, CH)], rows0_v)
        pltpu.sync_copy(rows0_v, out_hbm.at[c, pl.ds(off, CH)])
        return carry

    lax.fori_loop(0, RCH, obody, 0)


@functools.lru_cache(maxsize=1)
def _sc_segsum_kernel():
    return pl.kernel(
        _sc_segsum_body,
        out_type=jax.ShapeDtypeStruct((NC, N_PAD, D), jnp.float32),
        mesh=plsc.VectorSubcoreMesh(
            core_axis_name="c", subcore_axis_name="s",
            num_cores=NC, num_subcores=NS),
        scratch_types=[
            pltpu.VMEM((EPW,), jnp.int32),
            pltpu.VMEM((NCHUNK, CH), jnp.int32),
            pltpu.VMEM((CH, D), jnp.float32),
            pltpu.VMEM((CH, D), jnp.float32),
            pltpu.VMEM_SHARED((N_PAD, D), jnp.float32),
            pltpu.SemaphoreType.DMA,
            pltpu.SemaphoreType.DMA,
            pltpu.SemaphoreType.DMA,
            pltpu.SemaphoreType.DMA,
            pltpu.SemaphoreType.DMA,
        ],
    )


def _sc_segsum(h, src, dstc, zeros):
    return _sc_segsum_kernel()(h, src, dstc, zeros)


def _mlp_body(h_ref, p0_ref, p1_ref, w1_ref, b1_ref, w2_ref, b2_ref,
              z2_ref, st_ref):
    i = pl.program_id(0)
    z = h_ref[...] + p0_ref[...] + p1_ref[...]
    a = jnp.maximum(
        jnp.dot(z, w1_ref[...], preferred_element_type=jnp.float32)
        + b1_ref[...], 0.0)
    z2 = (jnp.dot(a, w2_ref[...], preferred_element_type=jnp.float32)
          + b2_ref[...])
    z2_ref[...] = z2
    s0 = jnp.sum(z2, axis=0, keepdims=True)
    s1 = jnp.sum(z2 * z2, axis=0, keepdims=True)
    contrib = jnp.concatenate(
        [s0, s1, jnp.zeros((6, D), jnp.float32)], axis=0)

    @pl.when(i == 0)
    def _():
        st_ref[...] = contrib

    @pl.when(i > 0)
    def _():
        st_ref[...] = st_ref[...] + contrib


def _mlp(h, p0, p1, W1, b1, W2, b2):
    return pl.pallas_call(
        _mlp_body,
        grid=(NBLK,),
        in_specs=[
            pl.BlockSpec((RB, D), lambda i: (i, 0)),
            pl.BlockSpec((RB, D), lambda i: (i, 0)),
            pl.BlockSpec((RB, D), lambda i: (i, 0)),
            pl.BlockSpec((D, 2 * D), lambda i: (0, 0)),
            pl.BlockSpec((1, 2 * D), lambda i: (0, 0)),
            pl.BlockSpec((2 * D, D), lambda i: (0, 0)),
            pl.BlockSpec((1, D), lambda i: (0, 0)),
        ],
        out_specs=[
            pl.BlockSpec((RB, D), lambda i: (i, 0)),
            pl.BlockSpec((8, D), lambda i: (0, 0)),
        ],
        out_shape=[
            jax.ShapeDtypeStruct((N, D), jnp.float32),
            jax.ShapeDtypeStruct((8, D), jnp.float32),
        ],
    )(h, p0, p1, W1, b1.reshape(1, 2 * D), W2, b2.reshape(1, D))


def _bnrelu_body(z_ref, sc_ref, sh_ref, o_ref):
    o_ref[...] = jnp.maximum(z_ref[...] * sc_ref[...] + sh_ref[...], 0.0)


def _bnrelu(z, scale, shift):
    return pl.pallas_call(
        _bnrelu_body,
        grid=(NBLK,),
        in_specs=[
            pl.BlockSpec((RB, D), lambda i: (i, 0)),
            pl.BlockSpec((1, D), lambda i: (0, 0)),
            pl.BlockSpec((1, D), lambda i: (0, 0)),
        ],
        out_specs=pl.BlockSpec((RB, D), lambda i: (i, 0)),
        out_shape=jax.ShapeDtypeStruct((N, D), jnp.float32),
    )(z, scale, shift)


def _segmax_body(starts_ref, blo_ref, bhi_ref, z_ref, sc_ref, sh_ref, out_ref):
    b = pl.program_id(0)

    @pl.when(b == 0)
    def _():
        out_ref[...] = jnp.full((G, D), -jnp.inf, jnp.float32)

    zn = z_ref[...] * sc_ref[...] + sh_ref[...]
    gi = b * RB + lax.broadcasted_iota(jnp.int32, (RB, D), 0)
    gidx = lax.broadcasted_iota(jnp.int32, (G, D), 0)

    def gbody(g, carry):
        lo = starts_ref[g]
        hi = starts_ref[g + 1]
        m = (gi >= lo) & (gi < hi)
        v = jnp.max(jnp.where(m, zn, -jnp.inf), axis=0, keepdims=True)
        upd = jnp.where(gidx == g, v, -jnp.inf)
        out_ref[...] = jnp.maximum(out_ref[...], upd)
        return carry

    lax.fori_loop(blo_ref[b], bhi_ref[b] + 1, gbody, 0)


def _segmax(z, scale, shift, starts, blo, bhi):
    return pl.pallas_call(
        _segmax_body,
        grid=(NBLK,),
        in_specs=[
            pl.BlockSpec(memory_space=pltpu.SMEM),
            pl.BlockSpec(memory_space=pltpu.SMEM),
            pl.BlockSpec(memory_space=pltpu.SMEM),
            pl.BlockSpec((RB, D), lambda i: (i, 0)),
            pl.BlockSpec((1, D), lambda i: (0, 0)),
            pl.BlockSpec((1, D), lambda i: (0, 0)),
        ],
        out_specs=pl.BlockSpec((G, D), lambda i: (0, 0)),
        out_shape=jax.ShapeDtypeStruct((G, D), jnp.float32),
    )(starts, blo, bhi, z, scale, shift)


def kernel(x, edge_index, batch,
           W1_0, b1_0, W2_0, b2_0, gamma_0, beta_0,
           W1_1, b1_1, W2_1, b2_1, gamma_1, beta_1):
    src = edge_index[0]
    dstc = edge_index[1].reshape(NW, NCHUNK, CH)
    zeros = jnp.zeros((CH, D), jnp.float32)
    starts = jnp.searchsorted(
        batch, jnp.arange(G + 1, dtype=jnp.int32), side='left'
    ).astype(jnp.int32)
    bidx = jnp.arange(NBLK, dtype=jnp.int32)
    blo = batch[bidx * RB]
    bhi = batch[bidx * RB + RB - 1]

    W1s = jnp.stack([W1_0, W1_1])
    b1s = jnp.stack([b1_0, b1_1])
    W2s = jnp.stack([W2_0, W2_1])
    b2s = jnp.stack([b2_0, b2_1])
    gammas = jnp.stack([gamma_0, gamma_1])
    betas = jnp.stack([beta_0, beta_1])

    def layer(h, p):
        W1, b1, W2, b2, gamma, beta = p
        parts = _sc_segsum(h, src, dstc, zeros)
        z2, st = _mlp(h, parts[0, :N], parts[1, :N], W1, b1, W2, b2)
        mean = st[0] / N
        var = st[1] / N - mean * mean
        scale = (gamma * lax.rsqrt(var + BN_EPS)).reshape(1, D)
        shift = (beta - mean * scale[0]).reshape(1, D)
        h_next = _bnrelu(z2, scale, shift)
        return h_next, (z2, scale, shift)

    _, (z2s, scales, shifts) = lax.scan(
        layer, x, (W1s, b1s, W2s, b2s, gammas, betas))

    return _segmax(z2s[1], scales[1], shifts[1], starts, blo, bhi)


# speedup vs baseline: 7.7175x; 1.1456x over previous
import functools

import jax
import jax.numpy as jnp
from jax import lax
from jax.experimental import pallas as pl
from jax.experimental.pallas import tpu as pltpu
from jax.experimental.pallas import tpu_sc as plsc

N = 10000
E = 320000
D = 128
G = 64
BN_EPS = 1e-5

NC = 2
NS = 16
NW = NC * NS
EPW = E // NW
CH = 80
NCHUNK = EPW // CH
N_PAD = 10240
RPT = N_PAD // NS
RCH = RPT // CH

RB = 400
NBLK = N // RB


def _sc_segsum_body(h_hbm, src_hbm, dstc_hbm, zeros_hbm, out_hbm,
                    src_v, dst_v, rows0_v, rows1_v, acc_sh,
                    isem, gsem0, gsem1, ssem0, ssem1):
    c = lax.axis_index("c")
    s = lax.axis_index("s")
    wid = s * NC + c

    rows = (rows0_v, rows1_v)
    gsem = (gsem0, gsem1)
    ssem = (ssem0, ssem1)

    src_view = src_hbm.at[pl.ds(pl.multiple_of(wid * EPW, 8), EPW)]
    dst_view = dstc_hbm.at[wid]
    pltpu.async_copy(src_view, src_v, isem)
    pltpu.async_copy(dst_view, dst_v, isem)

    pltpu.sync_copy(zeros_hbm, rows0_v)

    def zbody(j, carry):
        pltpu.sync_copy(rows0_v, acc_sh.at[pl.ds(s * RPT + j * CH, CH)])
        return carry

    lax.fori_loop(0, RCH, zbody, 0)
    pltpu.make_async_copy(src_view, src_v, isem).wait()
    pltpu.make_async_copy(dst_view, dst_v, isem).wait()
    plsc.subcore_barrier()

    def gather(i, p):
        pltpu.async_copy(h_hbm.at[src_v.at[pl.ds(i * CH, CH)]],
                         rows[p], gsem[p])

    def scat_start(i, p):
        pltpu.async_copy(rows[p], acc_sh.at[dst_v.at[i]], ssem[p],
                         add=True)

    def scat_wait(p):
        pltpu.make_async_copy(rows[p], acc_sh.at[pl.ds(0, CH)],
                              ssem[p]).wait()

    def gat_wait(p):
        pltpu.make_async_copy(h_hbm.at[pl.ds(0, CH)], rows[p],
                              gsem[p]).wait()

    gather(0, 0)
    gather(1, 1)

    def body(ii, carry):
        i = ii * 2
        gat_wait(0)
        scat_start(i, 0)
        gat_wait(1)
        scat_start(i + 1, 1)
        scat_wait(0)
        gather(i + 2, 0)
        scat_wait(1)

        @pl.when(i + 3 < NCHUNK)
        def _():
            gather(i + 3, 1)

        return carry

    lax.fori_loop(0, (NCHUNK - 1) // 2, body, 0)
    gat_wait(0)
    scat_start(NCHUNK - 1, 0)
    scat_wait(0)
    plsc.subcore_barrier()

    def obody(j, carry):
        off = s * RPT + j * CH
        pltpu.sync_copy(acc_sh.at[pl.ds(off, CH)], rows0_v)
        pltpu.sync_copy(rows0_v, out_hbm.at[c, pl.ds(off, CH)])
        return carry

    lax.fori_loop(0, RCH, obody, 0)


@functools.lru_cache(maxsize=1)
def _sc_segsum_kernel():
    return pl.kernel(
        _sc_segsum_body,
        out_type=jax.ShapeDtypeStruct((NC, N_PAD, D), jnp.float32),
        mesh=plsc.VectorSubcoreMesh(
            core_axis_name="c", subcore_axis_name="s",
            num_cores=NC, num_subcores=NS),
        scratch_types=[
            pltpu.VMEM((EPW,), jnp.int32),
            pltpu.VMEM((NCHUNK, CH), jnp.int32),
            pltpu.VMEM((CH, D), jnp.float32),
            pltpu.VMEM((CH, D), jnp.float32),
            pltpu.VMEM_SHARED((N_PAD, D), jnp.float32),
            pltpu.SemaphoreType.DMA,
            pltpu.SemaphoreType.DMA,
            pltpu.SemaphoreType.DMA,
            pltpu.SemaphoreType.DMA,
            pltpu.SemaphoreType.DMA,
        ],
    )


def _sc_segsum(h, src, dstc, zeros):
    return _sc_segsum_kernel()(h, src, dstc, zeros)


def _mlp_body(h_ref, p_ref, w1_ref, b1_ref, w2_ref, b2_ref,
              z2_ref, st_ref):
    i = pl.program_id(0)
    z = h_ref[...] + p_ref[0] + p_ref[1]
    a = jnp.maximum(
        jnp.dot(z, w1_ref[...], preferred_element_type=jnp.float32)
        + b1_ref[...], 0.0)
    z2 = (jnp.dot(a, w2_ref[...], preferred_element_type=jnp.float32)
          + b2_ref[...])
    z2_ref[...] = z2
    s0 = jnp.sum(z2, axis=0, keepdims=True)
    s1 = jnp.sum(z2 * z2, axis=0, keepdims=True)
    contrib = jnp.concatenate(
        [s0, s1, jnp.zeros((6, D), jnp.float32)], axis=0)

    @pl.when(i == 0)
    def _():
        st_ref[...] = contrib

    @pl.when(i > 0)
    def _():
        st_ref[...] = st_ref[...] + contrib


def _mlp(h, parts, W1, b1, W2, b2):
    return pl.pallas_call(
        _mlp_body,
        grid=(NBLK,),
        in_specs=[
            pl.BlockSpec((RB, D), lambda i: (i, 0)),
            pl.BlockSpec((NC, RB, D), lambda i: (0, i, 0)),
            pl.BlockSpec((D, 2 * D), lambda i: (0, 0)),
            pl.BlockSpec((1, 2 * D), lambda i: (0, 0)),
            pl.BlockSpec((2 * D, D), lambda i: (0, 0)),
            pl.BlockSpec((1, D), lambda i: (0, 0)),
        ],
        out_specs=[
            pl.BlockSpec((RB, D), lambda i: (i, 0)),
            pl.BlockSpec((8, D), lambda i: (0, 0)),
        ],
        out_shape=[
            jax.ShapeDtypeStruct((N, D), jnp.float32),
            jax.ShapeDtypeStruct((8, D), jnp.float32),
        ],
    )(h, parts, W1, b1.reshape(1, 2 * D), W2, b2.reshape(1, D))


def _bnrelu_body(z_ref, sc_ref, sh_ref, o_ref):
    o_ref[...] = jnp.maximum(z_ref[...] * sc_ref[...] + sh_ref[...], 0.0)


def _bnrelu(z, scale, shift):
    return pl.pallas_call(
        _bnrelu_body,
        grid=(NBLK,),
        in_specs=[
            pl.BlockSpec((RB, D), lambda i: (i, 0)),
            pl.BlockSpec((1, D), lambda i: (0, 0)),
            pl.BlockSpec((1, D), lambda i: (0, 0)),
        ],
        out_specs=pl.BlockSpec((RB, D), lambda i: (i, 0)),
        out_shape=jax.ShapeDtypeStruct((N, D), jnp.float32),
    )(z, scale, shift)


def _segmax_body(starts_ref, blo_ref, bhi_ref, z_ref, sc_ref, sh_ref, out_ref):
    b = pl.program_id(0)

    @pl.when(b == 0)
    def _():
        out_ref[...] = jnp.full((G, D), -jnp.inf, jnp.float32)

    zn = z_ref[...] * sc_ref[...] + sh_ref[...]
    gi = b * RB + lax.broadcasted_iota(jnp.int32, (RB, D), 0)
    gidx = lax.broadcasted_iota(jnp.int32, (G, D), 0)

    def gbody(g, carry):
        lo = starts_ref[g]
        hi = starts_ref[g + 1]
        m = (gi >= lo) & (gi < hi)
        v = jnp.max(jnp.where(m, zn, -jnp.inf), axis=0, keepdims=True)
        upd = jnp.where(gidx == g, v, -jnp.inf)
        out_ref[...] = jnp.maximum(out_ref[...], upd)
        return carry

    lax.fori_loop(blo_ref[b], bhi_ref[b] + 1, gbody, 0)


def _segmax(z, scale, shift, starts, blo, bhi):
    return pl.pallas_call(
        _segmax_body,
        grid=(NBLK,),
        in_specs=[
            pl.BlockSpec(memory_space=pltpu.SMEM),
            pl.BlockSpec(memory_space=pltpu.SMEM),
            pl.BlockSpec(memory_space=pltpu.SMEM),
            pl.BlockSpec((RB, D), lambda i: (i, 0)),
            pl.BlockSpec((1, D), lambda i: (0, 0)),
            pl.BlockSpec((1, D), lambda i: (0, 0)),
        ],
        out_specs=pl.BlockSpec((G, D), lambda i: (0, 0)),
        out_shape=jax.ShapeDtypeStruct((G, D), jnp.float32),
    )(starts, blo, bhi, z, scale, shift)


def kernel(x, edge_index, batch,
           W1_0, b1_0, W2_0, b2_0, gamma_0, beta_0,
           W1_1, b1_1, W2_1, b2_1, gamma_1, beta_1):
    src = edge_index[0]
    dstc = edge_index[1].reshape(NW, NCHUNK, CH)
    zeros = jnp.zeros((CH, D), jnp.float32)
    starts = jnp.searchsorted(
        batch, jnp.arange(G + 1, dtype=jnp.int32), side='left'
    ).astype(jnp.int32)
    bidx = jnp.arange(NBLK, dtype=jnp.int32)
    blo = batch[bidx * RB]
    bhi = batch[bidx * RB + RB - 1]

    params = [
        (W1_0, b1_0, W2_0, b2_0, gamma_0, beta_0),
        (W1_1, b1_1, W2_1, b2_1, gamma_1, beta_1),
    ]
    h = x
    for l, (W1, b1, W2, b2, gamma, beta) in enumerate(params):
        parts = _sc_segsum(h, src, dstc, zeros)
        z2, st = _mlp(h, parts, W1, b1, W2, b2)
        mean = st[0] / N
        var = st[1] / N - mean * mean
        scale = (gamma * lax.rsqrt(var + BN_EPS)).reshape(1, D)
        shift = (beta - mean * scale[0]).reshape(1, D)
        if l == 0:
            h = _bnrelu(z2, scale, shift)
        else:
            return _segmax(z2, scale, shift, starts, blo, bhi)
